# Initial kernel scaffold; baseline (speedup 1.0000x reference)
import jax
import jax.numpy as jnp
from jax.experimental import pallas as pl


def kernel(x, edge_index, src, dst, n_src, n_dst, g0_w1, g0_b1, g0_w2, g0_b2, g1_w1, g1_b1, g1_w2, g1_b2, g2_w1, g2_b1, g2_w2, g2_b2, jk_w, jk_b, o0_w, o0_b, o1_w, o1_b, o2_w, o2_b):
    raise NotImplementedError("write your pallas kernel here")


# speedup vs baseline: 5.0501x; 5.0501x over previous
import functools

import jax
import jax.numpy as jnp
from jax import lax
from jax.experimental import pallas as pl
from jax.experimental.pallas import tpu as pltpu
from jax.experimental.pallas import tpu_sc as plsc

N = 10000
D = 128
ZD = 2 * D
E = 320000
B = 16
MS = 48
MD = 96
NC = 2
NS = 16
NW = NC * NS
EW = E // NW
CH = 128
NFULL = EW // CH
TAIL = EW - NFULL * CH
RPT = 624
NTR = N - NS * RPT
GN = 2560
GW = GN // NW
NSD = 512 + 1024

@functools.cache
def _mesh():
    return plsc.VectorSubcoreMesh(
        core_axis_name="c", subcore_axis_name="s",
        num_cores=NC, num_subcores=NS)


@functools.cache
def _segsum_kernel():
    return functools.partial(
        pl.kernel,
        out_type=jax.ShapeDtypeStruct((NC * N, D), jnp.float32),
        mesh=_mesh(),
        scratch_types=[
            pltpu.VMEM((CH,), jnp.int32),
            pltpu.VMEM((CH,), jnp.int32),
            pltpu.VMEM((TAIL,), jnp.int32),
            pltpu.VMEM((TAIL,), jnp.int32),
            pltpu.VMEM((CH, D), jnp.float32),
            pltpu.VMEM((TAIL, D), jnp.float32),
            pltpu.VMEM_SHARED((N, D), jnp.float32),
            pltpu.SemaphoreType.DMA,
        ],
    )(_segsum_body)


def _segsum(h, es, ed, zero):
    return _segsum_kernel()(h, es, ed, zero)


def _segsum_body(h_hbm, es_hbm, ed_hbm, zero_hbm, out_hbm,
                 sidx, didx, sidx_t, didx_t, rows, rows_t, acc, sem):
    c = lax.axis_index("c")
    s = lax.axis_index("s")
    wid = c * NS + s
    r0 = s * RPT
    pltpu.sync_copy(zero_hbm.at[pl.ds(r0, RPT)], acc.at[pl.ds(r0, RPT)])

    @pl.when(s == NS - 1)
    def _():
        pltpu.sync_copy(zero_hbm.at[pl.ds(NS * RPT, NTR)],
                        acc.at[pl.ds(NS * RPT, NTR)])

    plsc.subcore_barrier()
    ebase = wid * EW

    @pl.loop(0, NFULL)
    def _(i):
        b = pl.multiple_of(ebase + i * CH, CH)
        pltpu.sync_copy(es_hbm.at[pl.ds(b, CH)], sidx)
        pltpu.sync_copy(ed_hbm.at[pl.ds(b, CH)], didx)
        pltpu.async_copy(h_hbm.at[sidx], rows, sem).wait()
        pltpu.sync_copy(rows, acc.at[didx], add=True)

    bt = ebase + NFULL * CH
    pltpu.sync_copy(es_hbm.at[pl.ds(bt, TAIL)], sidx_t)
    pltpu.sync_copy(ed_hbm.at[pl.ds(bt, TAIL)], didx_t)
    pltpu.async_copy(h_hbm.at[sidx_t], rows_t, sem).wait()
    pltpu.sync_copy(rows_t, acc.at[didx_t], add=True)
    plsc.subcore_barrier()
    pltpu.sync_copy(acc.at[pl.ds(r0, RPT)],
                    out_hbm.at[pl.ds(c * N + r0, RPT)])

    @pl.when(s == NS - 1)
    def _():
        pltpu.sync_copy(acc.at[pl.ds(NS * RPT, NTR)],
                        out_hbm.at[pl.ds(c * N + NS * RPT, NTR)])


@functools.cache
def _gatherz_kernel():
    return functools.partial(
        pl.kernel,
        out_type=jax.ShapeDtypeStruct((GN, ZD), jnp.float32),
        mesh=_mesh(),
        scratch_types=[
            pltpu.VMEM((GW,), jnp.int32),
            pltpu.VMEM((GW,), jnp.int32),
            pltpu.VMEM((GW, ZD), jnp.float32),
            pltpu.SemaphoreType.DMA,
        ],
    )(_gatherz_body)


def _gatherz(z, catsd, cidx):
    return _gatherz_kernel()(z, catsd, cidx)


def _gatherz_body(z_hbm, catsd_hbm, cidx_hbm, out_hbm,
                  cidx_v, gidx_v, rows_v, sem):
    c = lax.axis_index("c")
    s = lax.axis_index("s")
    wid = c * NS + s
    b = wid * GW
    pltpu.sync_copy(cidx_hbm.at[pl.ds(b, GW)], cidx_v)
    pltpu.async_copy(catsd_hbm.at[cidx_v], gidx_v, sem).wait()
    pltpu.async_copy(z_hbm.at[gidx_v], rows_v, sem).wait()
    pltpu.sync_copy(rows_v, out_hbm.at[pl.ds(b, GW)])


def _gin_block(a0, a1, h, w1, b1, w2, b2, o):
    m = a0[...] + a1[...] + h[...]
    t = jnp.maximum(
        jnp.dot(m, w1[...], preferred_element_type=jnp.float32) + b1[...], 0.0)
    o[...] = jnp.maximum(
        jnp.dot(t, w2[...], preferred_element_type=jnp.float32) + b2[...], 0.0)


_GR = 1000


def _gin_call(agg2, h, w1, b1, w2, b2):
    return pl.pallas_call(
        _gin_block,
        grid=(N // _GR,),
        in_specs=[
            pl.BlockSpec((_GR, D), lambda i: (i, 0)),
            pl.BlockSpec((_GR, D), lambda i: (i + N // _GR, 0)),
            pl.BlockSpec((_GR, D), lambda i: (i, 0)),
            pl.BlockSpec((D, D), lambda i: (0, 0)),
            pl.BlockSpec((1, D), lambda i: (0, 0)),
            pl.BlockSpec((D, D), lambda i: (0, 0)),
            pl.BlockSpec((1, D), lambda i: (0, 0)),
        ],
        out_specs=pl.BlockSpec((_GR, D), lambda i: (i, 0)),
        out_shape=jax.ShapeDtypeStruct((N, D), jnp.float32),
    )(agg2, agg2, h, w1, b1.reshape(1, D), w2, b2.reshape(1, D))


def _jk_block(x, h1, h2, h3, w1, w2, w3, bb, o):
    z = (jnp.dot(h1[...], w1[...], preferred_element_type=jnp.float32)
         + jnp.dot(h2[...], w2[...], preferred_element_type=jnp.float32)
         + jnp.dot(h3[...], w3[...], preferred_element_type=jnp.float32)
         + bb[...])
    full = jnp.concatenate([x[...], z], axis=1)
    ss = jnp.sum(full * full, axis=1, keepdims=True)
    o[...] = full * lax.rsqrt(ss)


def _jk_call(x, h1, h2, h3, jk_w, jk_b):
    return pl.pallas_call(
        _jk_block,
        grid=(N // _GR,),
        in_specs=[pl.BlockSpec((_GR, D), lambda i: (i, 0))] * 4
        + [pl.BlockSpec((D, D), lambda i: (0, 0))] * 3
        + [pl.BlockSpec((1, D), lambda i: (0, 0))],
        out_specs=pl.BlockSpec((_GR, ZD), lambda i: (i, 0)),
        out_shape=jax.ShapeDtypeStruct((N, ZD), jnp.float32),
    )(x, h1, h2, h3, jk_w[:D], jk_w[D:2 * D], jk_w[2 * D:], jk_b.reshape(1, D))


def _pairs_block(sz, dz, pa, w0, b0, w1, b1, w2, o):
    comb = (sz[...][:, None, :] * dz[...][None, :, :]).reshape(MS * MD, ZD)
    u = jnp.maximum(
        jnp.dot(comb, w0[...], preferred_element_type=jnp.float32) + b0[...], 0.0)
    u = jnp.maximum(
        jnp.dot(u, w1[...], preferred_element_type=jnp.float32) + b1[...], 0.0)
    o[...] = jnp.dot(u, w2[...], preferred_element_type=jnp.float32) + pa[...]


def _pairs_call(rows, pa, o0_w, o0_b, o1_w, o1_b, o2_w):
    return pl.pallas_call(
        _pairs_block,
        grid=(B,),
        in_specs=[
            pl.BlockSpec((MS, ZD), lambda i: (i, 0)),
            pl.BlockSpec((MD, ZD), lambda i: (i + B * MS // MD, 0)),
            pl.BlockSpec((MS * MD, 1), lambda i: (i, 0)),
            pl.BlockSpec((ZD, ZD), lambda i: (0, 0)),
            pl.BlockSpec((1, ZD), lambda i: (0, 0)),
            pl.BlockSpec((ZD, D), lambda i: (0, 0)),
            pl.BlockSpec((1, D), lambda i: (0, 0)),
            pl.BlockSpec((D, 1), lambda i: (0, 0)),
        ],
        out_specs=pl.BlockSpec((MS * MD, 1), lambda i: (i, 0)),
        out_shape=jax.ShapeDtypeStruct((B * MS * MD, 1), jnp.float32),
    )(rows, rows, pa, o0_w, o0_b.reshape(1, ZD), o1_w, o1_b.reshape(1, D), o2_w)


def kernel(x, edge_index, src, dst, n_src, n_dst,
           g0_w1, g0_b1, g0_w2, g0_b2,
           g1_w1, g1_b1, g1_w2, g1_b2,
           g2_w1, g2_b1, g2_w2, g2_b2,
           jk_w, jk_b, o0_w, o0_b, o1_w, o1_b, o2_w, o2_b):
    es = edge_index[0]
    ed = edge_index[1]
    zero = jnp.zeros((N, D), jnp.float32)

    a = _segsum(x, es, ed, zero)
    h1 = _gin_call(a, x, g0_w1, g0_b1, g0_w2, g0_b2)
    a = _segsum(h1, es, ed, zero)
    h2 = _gin_call(a, h1, g1_w1, g1_b1, g1_w2, g1_b2)
    a = _segsum(h2, es, ed, zero)
    h3 = _gin_call(a, h2, g2_w1, g2_b1, g2_w2, g2_b2)

    z = _jk_call(x, h1, h2, h3, jk_w, jk_b)

    offs_s = jnp.concatenate(
        [jnp.zeros((1,), n_src.dtype), jnp.cumsum(n_src)[:-1]])
    offs_d = jnp.concatenate(
        [jnp.zeros((1,), n_dst.dtype), jnp.cumsum(n_dst)[:-1]])
    pos_s = jnp.arange(MS, dtype=jnp.int32)
    pos_d = jnp.arange(MD, dtype=jnp.int32)
    ci_s = jnp.clip(offs_s[:, None] + pos_s[None, :], 0, src.shape[0] - 1)
    ci_d = (jnp.clip(offs_d[:, None] + pos_d[None, :], 0, dst.shape[0] - 1)
            + src.shape[0])
    cidx = jnp.concatenate([ci_s.reshape(-1), ci_d.reshape(-1)])
    cidx = jnp.concatenate(
        [cidx, jnp.zeros((GN - cidx.shape[0],), jnp.int32)]).astype(jnp.int32)
    catsd = jnp.concatenate([src, dst]).astype(jnp.int32)

    rows = _gatherz(z, catsd, cidx)

    smask = pos_s[None, :] >= n_src[:, None]
    dmask = pos_d[None, :] >= n_dst[:, None]
    pmask = smask[:, :, None] | dmask[:, None, :]
    pa = jnp.where(pmask, -jnp.inf, o2_b[0]).astype(jnp.float32)
    pa = pa.reshape(B * MS * MD, 1)

    v = _pairs_call(rows, pa, o0_w, o0_b, o1_w, o1_b, o2_w)
    return v.reshape(B, MS * MD), MS, MD
